# baseline (speedup 1.0000x reference)
import functools

import numpy as np
import jax
import jax.numpy as jnp
from jax import lax
from jax.experimental import pallas as pl
from jax.experimental.pallas import tpu as pltpu
from jax.experimental.pallas import tpu_sc as plsc

D_MODEL = 768
MAX_LEN = 5000

NUM_CORES = 2
NUM_SUBCORES = 16
NUM_WORKERS = NUM_CORES * NUM_SUBCORES
LANES = 16


def _pe_table(time_steps: int) -> np.ndarray:
    half_dim = D_MODEL // 2
    pe = np.zeros((D_MODEL, MAX_LEN), dtype=np.float64)
    pos = np.arange(MAX_LEN)
    freq = 10000 ** (2 * np.arange(half_dim) / D_MODEL)
    pos_freq = pos.reshape((1, -1)) / freq.reshape((-1, 1))
    pe[:half_dim, :] = np.sin(pos_freq)
    pe[half_dim:, :] = np.cos(pos_freq)
    return pe.T[:time_steps].astype(np.float32)


@functools.partial(jax.jit, static_argnames=("batch", "seq_len"))
def _sc_embed(x_r, pe_bf, table, *, batch, seq_len):
    rows_total = batch * seq_len
    chunk = seq_len // NUM_WORKERS
    scale = float(np.sqrt(np.float32(D_MODEL)))
    pairs_per_row = D_MODEL // (2 * LANES)

    mesh = plsc.VectorSubcoreMesh(
        core_axis_name="c", subcore_axis_name="s")

    @functools.partial(
        pl.kernel,
        out_type=jax.ShapeDtypeStruct((rows_total, D_MODEL), jnp.float32),
        mesh=mesh,
        scratch_types=[
            pltpu.VMEM((batch, chunk), jnp.int32),
            pltpu.VMEM((chunk, D_MODEL // 2), jnp.int32),
            pltpu.VMEM((chunk, D_MODEL), jnp.float32),
            pltpu.VMEM((chunk, D_MODEL), jnp.float32),
            pltpu.SemaphoreType.DMA,
            pltpu.SemaphoreType.DMA,
            pltpu.SemaphoreType.DMA,
            pltpu.SemaphoreType.DMA,
            pltpu.SemaphoreType.DMA,
        ],
    )
    def k(xr_hbm, pe_hbm, table_hbm, out_hbm,
          idx_all, pe_v, rows0, rows1, gg0, gg1, gs0, gs1, gpe):
        rows_v = (rows0, rows1)
        sem_g = (gg0, gg1)
        sem_s = (gs0, gs1)

        wid = lax.axis_index("s") * NUM_CORES + lax.axis_index("c")
        t0 = wid * chunk

        pe_cp = pltpu.async_copy(pe_hbm.at[pl.ds(t0, chunk)], pe_v, gpe)
        pltpu.sync_copy(xr_hbm.at[wid], idx_all)

        def start_gather(b, buf):
            return pltpu.async_copy(
                table_hbm.at[idx_all.at[b]], rows_v[buf], sem_g[buf])

        def compute(buf):
            def row_body(r, _):
                for c2 in range(pairs_per_row):
                    w = pe_v[r, pl.ds(c2 * LANES, LANES)]
                    p0 = lax.bitcast_convert_type(w << 16, jnp.float32)
                    p1 = lax.bitcast_convert_type(
                        w & jnp.int32(-65536), jnp.float32)
                    sl0 = pl.ds((c2 * 2) * LANES, LANES)
                    sl1 = pl.ds((c2 * 2 + 1) * LANES, LANES)
                    rows_v[buf][r, sl0] = rows_v[buf][r, sl0] * scale + p0
                    rows_v[buf][r, sl1] = rows_v[buf][r, sl1] * scale + p1
                return 0
            lax.fori_loop(0, chunk, row_body, 0)

        for b in range(batch):
            start_gather(b, 0).wait()
            if b == 0:
                pe_cp.wait()
            compute(0)
            pltpu.async_copy(
                rows_v[0],
                out_hbm.at[pl.ds(b * seq_len + t0, chunk)],
                sem_s[0]).wait()

    return k(x_r, pe_bf, table)


def kernel(x, table):
    batch, seq_len = x.shape
    chunk = seq_len // NUM_WORKERS
    x_r = jnp.transpose(
        x.reshape(batch, NUM_WORKERS, chunk), (1, 0, 2))
    pe = _pe_table(seq_len)
    u32 = pe.view(np.uint32)
    rnd = ((u32 >> 16) & 1) + np.uint32(0x7FFF)
    bf = ((u32 + rnd) >> 16).astype(np.uint32)
    blk = bf.reshape(seq_len, D_MODEL // 32, 2, LANES)
    words = (blk[:, :, 0, :] | (blk[:, :, 1, :] << np.uint32(16)))
    pe_bf = jnp.asarray(
        words.reshape(seq_len, D_MODEL // 2).view(np.int32))
    out = _sc_embed(x_r, pe_bf, table, batch=batch, seq_len=seq_len)
    return out.reshape(batch, seq_len, D_MODEL)


# speedup vs baseline: 1.5231x; 1.5231x over previous
import functools

import numpy as np
import jax
import jax.numpy as jnp
from jax import lax
from jax.experimental import pallas as pl
from jax.experimental.pallas import tpu as pltpu
from jax.experimental.pallas import tpu_sc as plsc

D_MODEL = 768
MAX_LEN = 5000

NUM_CORES = 2
NUM_SUBCORES = 16
NUM_WORKERS = NUM_CORES * NUM_SUBCORES
LANES = 16
STEP_ROWS = 32
NBUF = 3


def _pe_table(time_steps: int) -> np.ndarray:
    half_dim = D_MODEL // 2
    pe = np.zeros((D_MODEL, MAX_LEN), dtype=np.float64)
    pos = np.arange(MAX_LEN)
    freq = 10000 ** (2 * np.arange(half_dim) / D_MODEL)
    pos_freq = pos.reshape((1, -1)) / freq.reshape((-1, 1))
    pe[:half_dim, :] = np.sin(pos_freq)
    pe[half_dim:, :] = np.cos(pos_freq)
    return pe.T[:time_steps].astype(np.float32)


@functools.partial(jax.jit, static_argnames=("batch", "seq_len"))
def _sc_embed(x_r, pe, table, *, batch, seq_len):
    rows_total = batch * seq_len
    chunk = seq_len // NUM_WORKERS
    halves = chunk // STEP_ROWS
    steps = halves * batch
    scale = float(np.sqrt(np.float32(D_MODEL)))
    vregs_per_row = D_MODEL // LANES

    mesh = plsc.VectorSubcoreMesh(
        core_axis_name="c", subcore_axis_name="s")

    @functools.partial(
        pl.kernel,
        out_type=jax.ShapeDtypeStruct((rows_total, D_MODEL), jnp.float32),
        mesh=mesh,
        scratch_types=[
            pltpu.VMEM((steps, STEP_ROWS), jnp.int32),
            pltpu.VMEM((STEP_ROWS, D_MODEL), jnp.float32),
            pltpu.VMEM((STEP_ROWS, D_MODEL), jnp.float32),
            pltpu.VMEM((STEP_ROWS, D_MODEL), jnp.float32),
            pltpu.VMEM((STEP_ROWS, D_MODEL), jnp.float32),
            pltpu.VMEM((STEP_ROWS, D_MODEL), jnp.float32),
            pltpu.SemaphoreType.DMA,
            pltpu.SemaphoreType.DMA,
            pltpu.SemaphoreType.DMA,
            pltpu.SemaphoreType.DMA,
            pltpu.SemaphoreType.DMA,
            pltpu.SemaphoreType.DMA,
            pltpu.SemaphoreType.DMA,
            pltpu.SemaphoreType.DMA,
        ],
    )
    def k(xr_hbm, pe_hbm, table_hbm, out_hbm,
          idx_all, pea, peb, rows0, rows1, rows2,
          gpa, gpb, gg0, gg1, gg2, gs0, gs1, gs2):
        pe_v = (pea, peb)
        rows_v = (rows0, rows1, rows2)
        sem_g = (gg0, gg1, gg2)
        sem_s = (gs0, gs1, gs2)

        wid = lax.axis_index("s") * NUM_CORES + lax.axis_index("c")
        t0 = wid * chunk

        pe_cp = [
            pltpu.async_copy(
                pe_hbm.at[pl.ds(t0 + h * STEP_ROWS, STEP_ROWS)],
                pe_v[h], (gpa, gpb)[h])
            for h in range(halves)
        ]
        pltpu.sync_copy(xr_hbm.at[wid], idx_all)

        def start_gather(s, buf):
            return pltpu.async_copy(
                table_hbm.at[idx_all.at[s]], rows_v[buf], sem_g[buf])

        def compute(buf, half):
            def row_body(r, _):
                for c in range(vregs_per_row):
                    sl = pl.ds(c * LANES, LANES)
                    rows_v[buf][r, sl] = (
                        rows_v[buf][r, sl] * scale + pe_v[half][r, sl])
                return 0
            lax.fori_loop(0, STEP_ROWS, row_body, 0)

        gather = [None] * NBUF
        stores = [None] * NBUF
        for s0 in range(min(2, steps)):
            gather[s0 % NBUF] = start_gather(s0, s0 % NBUF)
        for s in range(steps):
            buf = s % NBUF
            half, b = divmod(s, batch)
            if s + 2 < steps:
                nbuf = (s + 2) % NBUF
                if stores[nbuf] is not None:
                    stores[nbuf].wait()
                    stores[nbuf] = None
                gather[nbuf] = start_gather(s + 2, nbuf)
            gather[buf].wait()
            if s == 0:
                pe_cp[0].wait()
            if s == batch:
                pe_cp[1].wait()
            compute(buf, half)
            stores[buf] = pltpu.async_copy(
                rows_v[buf],
                out_hbm.at[pl.ds(b * seq_len + t0 + half * STEP_ROWS,
                                 STEP_ROWS)],
                sem_s[buf])
        for st in stores:
            if st is not None:
                st.wait()

    return k(x_r, pe, table)


def kernel(x, table):
    batch, seq_len = x.shape
    chunk = seq_len // NUM_WORKERS
    halves = chunk // STEP_ROWS
    x_r = jnp.transpose(
        x.reshape(batch, NUM_WORKERS, halves, STEP_ROWS),
        (1, 2, 0, 3)).reshape(NUM_WORKERS, halves * batch, STEP_ROWS)
    pe = jnp.asarray(_pe_table(seq_len))
    out = _sc_embed(x_r, pe, table, batch=batch, seq_len=seq_len)
    return out.reshape(batch, seq_len, D_MODEL)


# speedup vs baseline: 1.5316x; 1.0056x over previous
import functools

import numpy as np
import jax
import jax.numpy as jnp
from jax import lax
from jax.experimental import pallas as pl
from jax.experimental.pallas import tpu as pltpu
from jax.experimental.pallas import tpu_sc as plsc

D_MODEL = 768
MAX_LEN = 5000

NUM_CORES = 2
NUM_SUBCORES = 16
NUM_WORKERS = NUM_CORES * NUM_SUBCORES
LANES = 16
STEP_ROWS = 32
NBUF = 3


def _pe_table(time_steps: int) -> np.ndarray:
    half_dim = D_MODEL // 2
    pe = np.zeros((D_MODEL, MAX_LEN), dtype=np.float64)
    pos = np.arange(MAX_LEN)
    freq = 10000 ** (2 * np.arange(half_dim) / D_MODEL)
    pos_freq = pos.reshape((1, -1)) / freq.reshape((-1, 1))
    pe[:half_dim, :] = np.sin(pos_freq)
    pe[half_dim:, :] = np.cos(pos_freq)
    return pe.T[:time_steps].astype(np.float32)


@functools.partial(jax.jit, static_argnames=("batch", "seq_len"))
def _sc_embed(x_r, pe, table, *, batch, seq_len):
    rows_total = batch * seq_len
    chunk = seq_len // NUM_WORKERS
    halves = chunk // STEP_ROWS
    steps = halves * batch
    scale = float(np.sqrt(np.float32(D_MODEL)))
    vregs_per_row = D_MODEL // LANES

    mesh = plsc.VectorSubcoreMesh(
        core_axis_name="c", subcore_axis_name="s")

    @functools.partial(
        pl.kernel,
        out_type=jax.ShapeDtypeStruct((rows_total, D_MODEL), jnp.float32),
        mesh=mesh,
        scratch_types=[
            pltpu.VMEM((steps, STEP_ROWS), jnp.int32),
            pltpu.VMEM((STEP_ROWS, D_MODEL), jnp.float32),
            pltpu.VMEM((STEP_ROWS, D_MODEL), jnp.float32),
            pltpu.VMEM((STEP_ROWS, D_MODEL), jnp.float32),
            pltpu.VMEM((STEP_ROWS, D_MODEL), jnp.float32),
            pltpu.VMEM((STEP_ROWS, D_MODEL), jnp.float32),
            pltpu.SemaphoreType.DMA,
            pltpu.SemaphoreType.DMA,
            pltpu.SemaphoreType.DMA,
            pltpu.SemaphoreType.DMA,
            pltpu.SemaphoreType.DMA,
            pltpu.SemaphoreType.DMA,
            pltpu.SemaphoreType.DMA,
            pltpu.SemaphoreType.DMA,
            pltpu.SemaphoreType.DMA,
        ],
    )
    def k(xr_hbm, pe_hbm, table_hbm, out_hbm,
          idx_all, pea, peb, rows0, rows1, rows2,
          gpa, gpb, gg0, gg1, gg2, gs0, gs1, gs2, gsx):
        pe_v = (pea, peb)
        rows_v = (rows0, rows1, rows2)
        sem_g = (gg0, gg1, gg2)
        sem_s = (gs0, gs1, gs2)

        wid = lax.axis_index("s") * NUM_CORES + lax.axis_index("c")
        t0 = wid * chunk

        pe_cp = [
            pltpu.async_copy(
                pe_hbm.at[pl.ds(t0 + h * STEP_ROWS, STEP_ROWS)],
                pe_v[h], (gpa, gpb)[h])
            for h in range(halves)
        ]
        idx_cp = []
        for h in range(halves):
            for b in range(batch):
                s = h * batch + b
                idx_cp.append(pltpu.async_copy(
                    xr_hbm.at[pl.ds(b * seq_len + t0 + h * STEP_ROWS,
                                    STEP_ROWS)],
                    idx_all.at[s], gsx))
        for cp in idx_cp:
            cp.wait()

        def start_gather(s, buf):
            return pltpu.async_copy(
                table_hbm.at[idx_all.at[s]], rows_v[buf], sem_g[buf])

        def compute(buf, half):
            def row_body(r, _):
                for c in range(vregs_per_row):
                    sl = pl.ds(c * LANES, LANES)
                    rows_v[buf][r, sl] = (
                        rows_v[buf][r, sl] * scale + pe_v[half][r, sl])
                return 0
            lax.fori_loop(0, STEP_ROWS, row_body, 0)

        gather = [None] * NBUF
        stores = [None] * NBUF
        for s0 in range(min(2, steps)):
            gather[s0 % NBUF] = start_gather(s0, s0 % NBUF)
        for s in range(steps):
            buf = s % NBUF
            half, b = divmod(s, batch)
            if s + 2 < steps:
                nbuf = (s + 2) % NBUF
                if stores[nbuf] is not None:
                    stores[nbuf].wait()
                    stores[nbuf] = None
                gather[nbuf] = start_gather(s + 2, nbuf)
            gather[buf].wait()
            if s == 0:
                pe_cp[0].wait()
            if s == batch:
                pe_cp[1].wait()
            compute(buf, half)
            stores[buf] = pltpu.async_copy(
                rows_v[buf],
                out_hbm.at[pl.ds(b * seq_len + t0 + half * STEP_ROWS,
                                 STEP_ROWS)],
                sem_s[buf])
        for st in stores:
            if st is not None:
                st.wait()

    return k(x_r, pe, table)


def kernel(x, table):
    batch, seq_len = x.shape
    pe = jnp.asarray(_pe_table(seq_len))
    out = _sc_embed(x.reshape(-1), pe, table, batch=batch, seq_len=seq_len)
    return out.reshape(batch, seq_len, D_MODEL)


# speedup vs baseline: 1.5420x; 1.0068x over previous
import functools

import numpy as np
import jax
import jax.numpy as jnp
from jax import lax
from jax.experimental import pallas as pl
from jax.experimental.pallas import tpu as pltpu
from jax.experimental.pallas import tpu_sc as plsc

D_MODEL = 768
MAX_LEN = 5000

NUM_CORES = 2
NUM_SUBCORES = 16
NUM_WORKERS = NUM_CORES * NUM_SUBCORES
LANES = 16
STEP_ROWS = 32
NBUF = 3


def _pe_table(time_steps: int) -> np.ndarray:
    half_dim = D_MODEL // 2
    pe = np.zeros((D_MODEL, MAX_LEN), dtype=np.float64)
    pos = np.arange(MAX_LEN)
    freq = 10000 ** (2 * np.arange(half_dim) / D_MODEL)
    pos_freq = pos.reshape((1, -1)) / freq.reshape((-1, 1))
    pe[:half_dim, :] = np.sin(pos_freq)
    pe[half_dim:, :] = np.cos(pos_freq)
    return pe.T[:time_steps].astype(np.float32)


@functools.partial(jax.jit, static_argnames=("batch", "seq_len"))
def _sc_embed(x_r, pe, table, *, batch, seq_len):
    rows_total = batch * seq_len
    chunk = seq_len // NUM_WORKERS
    halves = chunk // STEP_ROWS
    steps = halves * batch
    scale = float(np.sqrt(np.float32(D_MODEL)))
    vregs_per_row = D_MODEL // LANES

    mesh = plsc.VectorSubcoreMesh(
        core_axis_name="c", subcore_axis_name="s")

    @functools.partial(
        pl.kernel,
        out_type=jax.ShapeDtypeStruct((rows_total, D_MODEL), jnp.float32),
        mesh=mesh,
        scratch_types=[
            pltpu.VMEM((steps, STEP_ROWS), jnp.int32),
            pltpu.VMEM((STEP_ROWS, D_MODEL), jnp.float32),
            pltpu.VMEM((STEP_ROWS, D_MODEL), jnp.float32),
            pltpu.VMEM((STEP_ROWS, D_MODEL), jnp.float32),
            pltpu.VMEM((STEP_ROWS, D_MODEL), jnp.float32),
            pltpu.VMEM((STEP_ROWS, D_MODEL), jnp.float32),
            pltpu.SemaphoreType.DMA,
            pltpu.SemaphoreType.DMA,
            pltpu.SemaphoreType.DMA,
            pltpu.SemaphoreType.DMA,
            pltpu.SemaphoreType.DMA,
            pltpu.SemaphoreType.DMA,
            pltpu.SemaphoreType.DMA,
            pltpu.SemaphoreType.DMA,
            pltpu.SemaphoreType.DMA,
        ],
    )
    def k(xr_hbm, pe_hbm, table_hbm, out_hbm,
          idx_all, pea, peb, rows0, rows1, rows2,
          gpa, gpb, gg0, gg1, gg2, gs0, gs1, gs2, gsx):
        pe_v = (pea, peb)
        rows_v = (rows0, rows1, rows2)
        sem_g = (gg0, gg1, gg2)
        sem_s = (gs0, gs1, gs2)

        wid = lax.axis_index("s") * NUM_CORES + lax.axis_index("c")
        t0 = wid * chunk

        def stage_idx(s):
            h, b = divmod(s, batch)
            return pltpu.async_copy(
                xr_hbm.at[pl.ds(b * seq_len + t0 + h * STEP_ROWS,
                                STEP_ROWS)],
                idx_all.at[s], gsx)

        first_idx = [stage_idx(s) for s in range(2)]
        for cp in first_idx:
            cp.wait()

        def start_gather(s, buf):
            return pltpu.async_copy(
                table_hbm.at[idx_all.at[s]], rows_v[buf], sem_g[buf])

        def compute(buf, half):
            def row_body(r, _):
                for c in range(vregs_per_row):
                    sl = pl.ds(c * LANES, LANES)
                    rows_v[buf][r, sl] = (
                        rows_v[buf][r, sl] * scale + pe_v[half][r, sl])
                return 0
            lax.fori_loop(0, STEP_ROWS, row_body, 0)

        gather = [None] * NBUF
        stores = [None] * NBUF
        for s0 in range(min(2, steps)):
            gather[s0 % NBUF] = start_gather(s0, s0 % NBUF)
        pe_cp = [
            pltpu.async_copy(
                pe_hbm.at[pl.ds(t0 + h * STEP_ROWS, STEP_ROWS)],
                pe_v[h], (gpa, gpb)[h])
            for h in range(halves)
        ]
        rest_idx = [stage_idx(s) for s in range(2, steps)]
        for s in range(steps):
            buf = s % NBUF
            half, b = divmod(s, batch)
            if s + 2 < steps:
                nbuf = (s + 2) % NBUF
                if stores[nbuf] is not None:
                    stores[nbuf].wait()
                    stores[nbuf] = None
                rest_idx[s].wait()
                gather[nbuf] = start_gather(s + 2, nbuf)
            gather[buf].wait()
            if s == 0:
                pe_cp[0].wait()
            if s == batch:
                pe_cp[1].wait()
            compute(buf, half)
            stores[buf] = pltpu.async_copy(
                rows_v[buf],
                out_hbm.at[pl.ds(b * seq_len + t0 + half * STEP_ROWS,
                                 STEP_ROWS)],
                sem_s[buf])
        for st in stores:
            if st is not None:
                st.wait()

    return k(x_r, pe, table)


def kernel(x, table):
    batch, seq_len = x.shape
    pe = jnp.asarray(_pe_table(seq_len))
    out = _sc_embed(x.reshape(-1), pe, table, batch=batch, seq_len=seq_len)
    return out.reshape(batch, seq_len, D_MODEL)
